# baseline (speedup 1.0000x reference)
import functools

import jax
import jax.numpy as jnp
from jax import lax
from jax.experimental import pallas as pl
from jax.experimental.pallas import tpu as pltpu
from jax.experimental.pallas import tpu_sc as plsc

NC = 2
NS = 16
NW = NC * NS
LANES = 16
CHUNK = 128


def _matmul_body(x_ref, w_ref, o_ref):
    o_ref[...] = jnp.dot(x_ref[...], w_ref[...],
                         preferred_element_type=jnp.float32)


def _support_matmul(x, W):
    N, D_in = x.shape
    D_out = W.shape[1]
    BM = 1000
    return pl.pallas_call(
        _matmul_body,
        grid=(N // BM,),
        in_specs=[pl.BlockSpec((BM, D_in), lambda i: (i, 0)),
                  pl.BlockSpec((D_in, D_out), lambda i: (0, 0))],
        out_specs=pl.BlockSpec((BM, D_out), lambda i: (i, 0)),
        out_shape=jax.ShapeDtypeStruct((N, D_out), jnp.float32),
    )(x, W)


def _add_body(a_ref, b_ref, o_ref):
    o_ref[...] = a_ref[...] + b_ref[...]


def _combine(p0, p1):
    N, D = p0.shape
    BM = 1000
    return pl.pallas_call(
        _add_body,
        grid=(N // BM,),
        in_specs=[pl.BlockSpec((BM, D), lambda i: (i, 0)),
                  pl.BlockSpec((BM, D), lambda i: (i, 0))],
        out_specs=pl.BlockSpec((BM, D), lambda i: (i, 0)),
        out_shape=jax.ShapeDtypeStruct((N, D), jnp.float32),
    )(p0, p1)


@functools.lru_cache(maxsize=None)
def _make_sc_scatter(N, D, K):
    mesh = plsc.VectorSubcoreMesh(core_axis_name="c", subcore_axis_name="s",
                                  num_cores=NC, num_subcores=NS)
    rpt = N // NS
    nfull = rpt // CHUNK
    rem = rpt - nfull * CHUNK

    @functools.partial(
        pl.kernel,
        out_type=jax.ShapeDtypeStruct((NC, N, D), jnp.float32),
        mesh=mesh,
        scratch_types=[
            pltpu.VMEM((K, CHUNK), jnp.int32),
            pltpu.VMEM((K, CHUNK), jnp.int32),
            pltpu.VMEM((K, CHUNK), jnp.float32),
            pltpu.VMEM((CHUNK, D), jnp.float32),
            pltpu.VMEM_SHARED((N, D), jnp.float32),
            pltpu.SemaphoreType.DMA,
        ],
    )
    def sc_scatter(support, src, dst, w, out, src_v, dst_v, w_v, rows,
                   acc, gsem):
        cid = lax.axis_index("c")
        sid = lax.axis_index("s")
        wid = cid * NS + sid

        pltpu.sync_copy(src.at[wid], src_v)
        pltpu.sync_copy(dst.at[wid], dst_v)
        pltpu.sync_copy(w.at[wid], w_v)

        zero16 = jnp.zeros((LANES,), jnp.float32)

        def zrow(r, carry):
            for c in range(D // LANES):
                rows[r, pl.ds(c * LANES, LANES)] = zero16
            return carry

        lax.fori_loop(0, CHUNK, zrow, 0)
        base = sid * rpt
        for j in range(nfull):
            pltpu.sync_copy(rows, acc.at[pl.ds(base + j * CHUNK, CHUNK)])
        if rem:
            pltpu.sync_copy(rows.at[pl.ds(0, rem)],
                            acc.at[pl.ds(base + nfull * CHUNK, rem)])
        plsc.subcore_barrier()

        def scale(buf, k):
            def scale_g(g, c2):
                wvec = w_v[k, pl.ds(g * LANES, LANES)]
                for l in range(LANES):
                    wt = wvec[l]
                    e = g * LANES + l
                    for c in range(D // LANES):
                        sl = pl.ds(c * LANES, LANES)
                        buf[e, sl] = buf[e, sl] * wt
                return c2

            lax.fori_loop(0, CHUNK // LANES, scale_g, 0)

        def chunk_body(k, carry):
            pltpu.async_copy(support.at[src_v.at[k]], rows, gsem).wait()
            scale(rows, k)
            pltpu.sync_copy(rows, acc.at[dst_v.at[k]], add=True)
            return carry

        lax.fori_loop(0, K, chunk_body, 0)

        plsc.subcore_barrier()
        pltpu.sync_copy(acc.at[pl.ds(base, rpt)],
                        out.at[cid, pl.ds(base, rpt)])

    return sc_scatter


def kernel(x, edge_index, edge_weight, W):
    N, _ = x.shape
    D = W.shape[1]
    E = edge_weight.shape[0]
    support = _support_matmul(x, W)

    align = NS * 8
    N_pad = -(-N // align) * align

    K = 4 * (-(-E // (NW * CHUNK * 4)))
    pad = NW * K * CHUNK - E
    src = jnp.concatenate(
        [edge_index[0], jnp.zeros((pad,), jnp.int32)]).reshape(NW, K, CHUNK)
    dst = jnp.concatenate(
        [edge_index[1], jnp.zeros((pad,), jnp.int32)]).reshape(NW, K, CHUNK)
    w = jnp.concatenate(
        [edge_weight, jnp.zeros((pad,), jnp.float32)]).reshape(NW, K, CHUNK)

    partials = _make_sc_scatter(N_pad, D, K)(support, src, dst, w)
    return _combine(partials[0, :N], partials[1, :N])


# speedup vs baseline: 2.1745x; 2.1745x over previous
import functools

import jax
import jax.numpy as jnp
from jax import lax
from jax.experimental import pallas as pl
from jax.experimental.pallas import tpu as pltpu
from jax.experimental.pallas import tpu_sc as plsc

NC = 2
NS = 16
NW = NC * NS
LANES = 16
CHUNK = 128


def _matmul_body(x_ref, w_ref, o_ref):
    o_ref[...] = jnp.dot(x_ref[...], w_ref[...],
                         preferred_element_type=jnp.float32)


def _support_matmul(x, W):
    N, D_in = x.shape
    D_out = W.shape[1]
    BM = 1000
    return pl.pallas_call(
        _matmul_body,
        grid=(N // BM,),
        in_specs=[pl.BlockSpec((BM, D_in), lambda i: (i, 0)),
                  pl.BlockSpec((D_in, D_out), lambda i: (0, 0))],
        out_specs=pl.BlockSpec((BM, D_out), lambda i: (i, 0)),
        out_shape=jax.ShapeDtypeStruct((N, D_out), jnp.float32),
    )(x, W)


def _add_body(a_ref, b_ref, o_ref):
    o_ref[...] = a_ref[...] + b_ref[...]


def _combine(p0, p1):
    N, D = p0.shape
    BM = 1000
    return pl.pallas_call(
        _add_body,
        grid=(N // BM,),
        in_specs=[pl.BlockSpec((BM, D), lambda i: (i, 0)),
                  pl.BlockSpec((BM, D), lambda i: (i, 0))],
        out_specs=pl.BlockSpec((BM, D), lambda i: (i, 0)),
        out_shape=jax.ShapeDtypeStruct((N, D), jnp.float32),
    )(p0, p1)


@functools.lru_cache(maxsize=None)
def _make_sc_scatter(N, D, K):
    mesh = plsc.VectorSubcoreMesh(core_axis_name="c", subcore_axis_name="s",
                                  num_cores=NC, num_subcores=NS)
    rpt = N // NS
    nfull = rpt // CHUNK
    rem = rpt - nfull * CHUNK

    @functools.partial(
        pl.kernel,
        out_type=jax.ShapeDtypeStruct((NC, N, D), jnp.float32),
        mesh=mesh,
        scratch_types=[
            pltpu.VMEM((K, CHUNK), jnp.int32),
            pltpu.VMEM((K, CHUNK), jnp.int32),
            pltpu.VMEM((K, CHUNK), jnp.float32),
            pltpu.VMEM((CHUNK, D), jnp.float32),
            pltpu.VMEM_SHARED((N, D), jnp.float32),
            pltpu.SemaphoreType.DMA,
        ],
    )
    def sc_scatter(support, src, dst, w, out, src_v, dst_v, w_v, rows,
                   acc, gsem):
        cid = lax.axis_index("c")
        sid = lax.axis_index("s")
        wid = cid * NS + sid

        pltpu.sync_copy(src.at[wid], src_v)
        pltpu.sync_copy(dst.at[wid], dst_v)
        pltpu.sync_copy(w.at[wid], w_v)

        zero16 = jnp.zeros((LANES,), jnp.float32)

        def zrow(r, carry):
            for c in range(D // LANES):
                rows[r, pl.ds(c * LANES, LANES)] = zero16
            return carry

        lax.fori_loop(0, CHUNK, zrow, 0)
        base = sid * rpt
        for j in range(nfull):
            pltpu.sync_copy(rows, acc.at[pl.ds(base + j * CHUNK, CHUNK)])
        if rem:
            pltpu.sync_copy(rows.at[pl.ds(0, rem)],
                            acc.at[pl.ds(base + nfull * CHUNK, rem)])
        plsc.subcore_barrier()

        def scale(buf, k):
            def scale_g(g, c2):
                wvec = w_v[k, pl.ds(g * LANES, LANES)]
                for l in range(LANES):
                    wt = wvec[l]
                    e = g * LANES + l
                    for c in range(D // LANES):
                        sl = pl.ds(c * LANES, LANES)
                        buf[e, sl] = buf[e, sl] * wt
                return c2

            lax.fori_loop(0, CHUNK // LANES, scale_g, 0)

        def chunk_body(k, carry):
            pltpu.async_copy(support.at[src_v.at[k]], rows, gsem).wait()
            scale(rows, k)
            pltpu.sync_copy(rows, acc.at[dst_v.at[k]], add=True)
            return carry

        lax.fori_loop(0, K, chunk_body, 0)

        plsc.subcore_barrier()
        pltpu.sync_copy(acc.at[pl.ds(base, rpt)],
                        out.at[cid, pl.ds(base, rpt)])

    return sc_scatter


def kernel(x, edge_index, edge_weight, W):
    N, _ = x.shape
    D = W.shape[1]
    E = edge_weight.shape[0]
    support = _support_matmul(x, W)

    align = NS * 8
    N_pad = -(-N // align) * align

    K = 4 * (-(-E // (NW * CHUNK * 4)))
    pad = NW * K * CHUNK - E
    pad_idx = (jnp.arange(pad, dtype=jnp.int32) * 8) % N
    src = jnp.concatenate(
        [edge_index[0], pad_idx]).reshape(NW, K, CHUNK)
    dst = jnp.concatenate(
        [edge_index[1], pad_idx]).reshape(NW, K, CHUNK)
    w = jnp.concatenate(
        [edge_weight, jnp.zeros((pad,), jnp.float32)]).reshape(NW, K, CHUNK)

    partials = _make_sc_scatter(N_pad, D, K)(support, src, dst, w)
    return _combine(partials[0, :N], partials[1, :N])


# speedup vs baseline: 3.1512x; 1.4491x over previous
import functools

import jax
import jax.numpy as jnp
from jax import lax
from jax.experimental import pallas as pl
from jax.experimental.pallas import tpu as pltpu
from jax.experimental.pallas import tpu_sc as plsc

NC = 2
NS = 16
NW = NC * NS
LANES = 16
CHUNK = 128


def _matmul_body(x_ref, w_ref, o_ref):
    o_ref[...] = jnp.dot(x_ref[...], w_ref[...],
                         preferred_element_type=jnp.float32)


def _support_matmul(x, W):
    N, D_in = x.shape
    D_out = W.shape[1]
    BM = 1000
    return pl.pallas_call(
        _matmul_body,
        grid=(N // BM,),
        in_specs=[pl.BlockSpec((BM, D_in), lambda i: (i, 0)),
                  pl.BlockSpec((D_in, D_out), lambda i: (0, 0))],
        out_specs=pl.BlockSpec((BM, D_out), lambda i: (i, 0)),
        out_shape=jax.ShapeDtypeStruct((N, D_out), jnp.float32),
    )(x, W)


def _add_body(a_ref, b_ref, o_ref):
    o_ref[...] = a_ref[...] + b_ref[...]


def _combine(p0, p1):
    N, D = p0.shape
    BM = 1000
    return pl.pallas_call(
        _add_body,
        grid=(N // BM,),
        in_specs=[pl.BlockSpec((BM, D), lambda i: (i, 0)),
                  pl.BlockSpec((BM, D), lambda i: (i, 0))],
        out_specs=pl.BlockSpec((BM, D), lambda i: (i, 0)),
        out_shape=jax.ShapeDtypeStruct((N, D), jnp.float32),
    )(p0, p1)


@functools.lru_cache(maxsize=None)
def _make_sc_scatter(N, D, K):
    mesh = plsc.VectorSubcoreMesh(core_axis_name="c", subcore_axis_name="s",
                                  num_cores=NC, num_subcores=NS)
    rpt = N // NS
    nfull = rpt // CHUNK
    rem = rpt - nfull * CHUNK

    @functools.partial(
        pl.kernel,
        out_type=jax.ShapeDtypeStruct((NC, N, D), jnp.float32),
        mesh=mesh,
        scratch_types=[
            pltpu.VMEM((K // 2, CHUNK), jnp.int32),
            pltpu.VMEM((K // 2, CHUNK), jnp.int32),
            pltpu.VMEM((K // 2, CHUNK), jnp.float32),
            pltpu.VMEM((CHUNK, D), jnp.float32),
            pltpu.VMEM((CHUNK, D), jnp.float32),
            pltpu.VMEM_SHARED((N, D), jnp.float32),
            pltpu.SemaphoreType.DMA,
        ],
    )
    def sc_scatter(support, src, dst, w, out, src_v, dst_v, w_v, rows, rows1,
                   acc, gsem):
        cid = lax.axis_index("c")
        sid = lax.axis_index("s")
        wid = cid * NS + sid
        HK = K // 2

        zero16 = jnp.zeros((LANES,), jnp.float32)

        def zrow(r, carry):
            for c in range(D // LANES):
                rows[r, pl.ds(c * LANES, LANES)] = zero16
            return carry

        lax.fori_loop(0, CHUNK, zrow, 0)
        base = sid * rpt
        for j in range(nfull):
            pltpu.sync_copy(rows, acc.at[pl.ds(base + j * CHUNK, CHUNK)])
        if rem:
            pltpu.sync_copy(rows.at[pl.ds(0, rem)],
                            acc.at[pl.ds(base + nfull * CHUNK, rem)])
        plsc.subcore_barrier()

        def scale(buf, k):
            def scale_g(g, c2):
                wvec = w_v[k, pl.ds(g * LANES, LANES)]
                for l in range(LANES):
                    wt = wvec[l]
                    e = g * LANES + l
                    for c in range(D // LANES):
                        sl = pl.ds(c * LANES, LANES)
                        buf[e, sl] = buf[e, sl] * wt
                return c2

            lax.fori_loop(0, CHUNK // LANES, scale_g, 0)

        for h in range(2):
            pltpu.sync_copy(src.at[wid, pl.ds(h * HK, HK)], src_v)
            pltpu.sync_copy(dst.at[wid, pl.ds(h * HK, HK)], dst_v)
            pltpu.sync_copy(w.at[wid, pl.ds(h * HK, HK)], w_v)
            pltpu.async_copy(support.at[src_v.at[0]], rows, gsem)

            def pair_body(i, carry):
                k0 = 2 * i
                k1 = k0 + 1
                pltpu.make_async_copy(support.at[src_v.at[k0]], rows,
                                      gsem).wait()
                pltpu.async_copy(support.at[src_v.at[k1]], rows1, gsem)
                scale(rows, k0)
                pltpu.sync_copy(rows, acc.at[dst_v.at[k0]], add=True)
                pltpu.make_async_copy(support.at[src_v.at[k1]], rows1,
                                      gsem).wait()

                @pl.when(i + 1 < HK // 2)
                def _():
                    pltpu.async_copy(support.at[src_v.at[k0 + 2]], rows, gsem)

                scale(rows1, k1)
                pltpu.sync_copy(rows1, acc.at[dst_v.at[k1]], add=True)
                return carry

            lax.fori_loop(0, HK // 2, pair_body, 0)

        plsc.subcore_barrier()
        pltpu.sync_copy(acc.at[pl.ds(base, rpt)],
                        out.at[cid, pl.ds(base, rpt)])

    return sc_scatter


def kernel(x, edge_index, edge_weight, W):
    N, _ = x.shape
    D = W.shape[1]
    E = edge_weight.shape[0]
    support = _support_matmul(x, W)

    align = NS * 8
    N_pad = -(-N // align) * align

    K = 4 * (-(-E // (NW * CHUNK * 4)))
    pad = NW * K * CHUNK - E
    pad_idx = (jnp.arange(pad, dtype=jnp.int32) * 8) % N
    src = jnp.concatenate(
        [edge_index[0], pad_idx]).reshape(NW, K, CHUNK)
    dst = jnp.concatenate(
        [edge_index[1], pad_idx]).reshape(NW, K, CHUNK)
    w = jnp.concatenate(
        [edge_weight, jnp.zeros((pad,), jnp.float32)]).reshape(NW, K, CHUNK)

    partials = _make_sc_scatter(N_pad, D, K)(support, src, dst, w)
    return _combine(partials[0, :N], partials[1, :N])


# speedup vs baseline: 3.1585x; 1.0023x over previous
import functools

import jax
import jax.numpy as jnp
from jax import lax
from jax.experimental import pallas as pl
from jax.experimental.pallas import tpu as pltpu
from jax.experimental.pallas import tpu_sc as plsc

NC = 2
NS = 16
NW = NC * NS
LANES = 16
CHUNK = 128


def _matmul_body(x_ref, w_ref, o_ref):
    o_ref[...] = jnp.dot(x_ref[...], w_ref[...],
                         preferred_element_type=jnp.float32)


def _support_matmul(x, W):
    N, D_in = x.shape
    D_out = W.shape[1]
    BM = 1000
    return pl.pallas_call(
        _matmul_body,
        grid=(N // BM,),
        in_specs=[pl.BlockSpec((BM, D_in), lambda i: (i, 0)),
                  pl.BlockSpec((D_in, D_out), lambda i: (0, 0))],
        out_specs=pl.BlockSpec((BM, D_out), lambda i: (i, 0)),
        out_shape=jax.ShapeDtypeStruct((N, D_out), jnp.float32),
    )(x, W)


def _add_body(a_ref, b_ref, o_ref):
    o_ref[...] = a_ref[...] + b_ref[...]


def _combine(p0, p1):
    N, D = p0.shape
    BM = 1000
    return pl.pallas_call(
        _add_body,
        grid=(N // BM,),
        in_specs=[pl.BlockSpec((BM, D), lambda i: (i, 0)),
                  pl.BlockSpec((BM, D), lambda i: (i, 0))],
        out_specs=pl.BlockSpec((BM, D), lambda i: (i, 0)),
        out_shape=jax.ShapeDtypeStruct((N, D), jnp.float32),
    )(p0, p1)


@functools.lru_cache(maxsize=None)
def _make_sc_scatter(N, D, K):
    mesh = plsc.VectorSubcoreMesh(core_axis_name="c", subcore_axis_name="s",
                                  num_cores=NC, num_subcores=NS)
    rpt = N // NS
    nfull = rpt // CHUNK
    rem = rpt - nfull * CHUNK

    @functools.partial(
        pl.kernel,
        out_type=jax.ShapeDtypeStruct((NC, N, D), jnp.float32),
        mesh=mesh,
        scratch_types=[
            pltpu.VMEM((K // 2, CHUNK), jnp.int32),
            pltpu.VMEM((K // 2, CHUNK), jnp.int32),
            pltpu.VMEM((K // 2, CHUNK), jnp.float32),
            pltpu.VMEM((CHUNK, D), jnp.float32),
            pltpu.VMEM((CHUNK, D), jnp.float32),
            pltpu.VMEM_SHARED((N, D), jnp.float32),
            pltpu.SemaphoreType.DMA,
            pltpu.SemaphoreType.DMA,
        ],
    )
    def sc_scatter(support, src, dst, w, out, src_v, dst_v, w_v, rows, rows1,
                   acc, gsem, ssem):
        cid = lax.axis_index("c")
        sid = lax.axis_index("s")
        wid = cid * NS + sid
        HK = K // 2

        zero16 = jnp.zeros((LANES,), jnp.float32)

        def zrow(r, carry):
            for c in range(D // LANES):
                rows[r, pl.ds(c * LANES, LANES)] = zero16
            return carry

        lax.fori_loop(0, CHUNK, zrow, 0)
        base = sid * rpt
        for j in range(nfull):
            pltpu.sync_copy(rows, acc.at[pl.ds(base + j * CHUNK, CHUNK)])
        if rem:
            pltpu.sync_copy(rows.at[pl.ds(0, rem)],
                            acc.at[pl.ds(base + nfull * CHUNK, rem)])
        plsc.subcore_barrier()

        def scale(buf, k):
            def scale_g(g, c2):
                wvec = w_v[k, pl.ds(g * LANES, LANES)]
                for l in range(LANES):
                    wt = wvec[l]
                    e = g * LANES + l
                    for c in range(D // LANES):
                        sl = pl.ds(c * LANES, LANES)
                        buf[e, sl] = buf[e, sl] * wt
                return c2

            lax.fori_loop(0, CHUNK // LANES, scale_g, 0)

        for h in range(2):
            pltpu.sync_copy(src.at[wid, pl.ds(h * HK, HK)], src_v)
            pltpu.sync_copy(dst.at[wid, pl.ds(h * HK, HK)], dst_v)
            pltpu.sync_copy(w.at[wid, pl.ds(h * HK, HK)], w_v)
            pltpu.async_copy(support.at[src_v.at[0]], rows, gsem)

            def pair_body(i, carry):
                k0 = 2 * i
                k1 = k0 + 1
                pltpu.make_async_copy(support.at[src_v.at[k0]], rows,
                                      gsem).wait()

                @pl.when(i > 0)
                def _():
                    pltpu.make_async_copy(rows1, acc.at[dst_v.at[k0 - 1]],
                                          ssem).wait()

                pltpu.async_copy(support.at[src_v.at[k1]], rows1, gsem)
                scale(rows, k0)
                sc0 = pltpu.async_copy(rows, acc.at[dst_v.at[k0]], ssem,
                                       add=True)
                pltpu.make_async_copy(support.at[src_v.at[k1]], rows1,
                                      gsem).wait()
                sc0.wait()

                @pl.when(i + 1 < HK // 2)
                def _():
                    pltpu.async_copy(support.at[src_v.at[k0 + 2]], rows, gsem)

                scale(rows1, k1)
                pltpu.async_copy(rows1, acc.at[dst_v.at[k1]], ssem, add=True)
                return carry

            lax.fori_loop(0, HK // 2, pair_body, 0)
            pltpu.make_async_copy(rows1, acc.at[dst_v.at[HK - 1]],
                                  ssem).wait()

        plsc.subcore_barrier()
        pltpu.sync_copy(acc.at[pl.ds(base, rpt)],
                        out.at[cid, pl.ds(base, rpt)])

    return sc_scatter


def kernel(x, edge_index, edge_weight, W):
    N, _ = x.shape
    D = W.shape[1]
    E = edge_weight.shape[0]
    support = _support_matmul(x, W)

    align = NS * 8
    N_pad = -(-N // align) * align

    K = 4 * (-(-E // (NW * CHUNK * 4)))
    pad = NW * K * CHUNK - E
    pad_idx = (jnp.arange(pad, dtype=jnp.int32) * 8) % N
    src = jnp.concatenate(
        [edge_index[0], pad_idx]).reshape(NW, K, CHUNK)
    dst = jnp.concatenate(
        [edge_index[1], pad_idx]).reshape(NW, K, CHUNK)
    w = jnp.concatenate(
        [edge_weight, jnp.zeros((pad,), jnp.float32)]).reshape(NW, K, CHUNK)

    partials = _make_sc_scatter(N_pad, D, K)(support, src, dst, w)
    return _combine(partials[0, :N], partials[1, :N])
